# baseline (speedup 1.0000x reference)
import functools

import jax
import jax.numpy as jnp
from jax import lax
from jax.experimental import pallas as pl
from jax.experimental.pallas import tpu as pltpu
from jax.experimental.pallas import tpu_sc as plsc

NC, NS = 2, 16
NW = NC * NS
CH = 128


def _gather_rows(table, idx, dim, nbuf=5):
    n = idx.shape[0]
    per_w = n // NW
    n_ch = per_w // CH
    n_grp = n_ch // nbuf
    assert n_ch % nbuf == 0 and n % (NW * CH) == 0
    mesh = plsc.VectorSubcoreMesh(
        core_axis_name="c", subcore_axis_name="s", num_cores=NC, num_subcores=NS
    )

    cp = pltpu.CompilerParams(use_tc_tiling_on_sc=(dim % 128 == 0))

    @functools.partial(
        pl.kernel,
        out_type=jax.ShapeDtypeStruct((n, dim), jnp.float32),
        mesh=mesh,
        compiler_params=cp,
        scratch_types=[
            pltpu.VMEM((per_w,), jnp.int32),
            [pltpu.VMEM((CH, dim), jnp.float32)] * nbuf,
            [pltpu.SemaphoreType.DMA] * nbuf,
            [pltpu.SemaphoreType.DMA] * nbuf,
        ],
    )
    def k(table_hbm, idx_hbm, out_hbm, idx_v, bufs, sem_g, sem_w):
        wid = lax.axis_index("s") * NC + lax.axis_index("c")
        w_base = wid * per_w

        pltpu.sync_copy(idx_hbm.at[pl.ds(w_base, per_w)], idx_v)

        def start_gather(b, j):
            pltpu.make_async_copy(
                table_hbm.at[idx_v.at[pl.ds(j * CH, CH)]], bufs[b], sem_g[b]
            ).start()

        def start_write(b, j):
            pltpu.make_async_copy(
                bufs[b], out_hbm.at[pl.ds(w_base + j * CH, CH)], sem_w[b]
            ).start()

        def wait_gather(b):
            pltpu.make_async_copy(
                table_hbm.at[idx_v.at[pl.ds(0, CH)]], bufs[b], sem_g[b]
            ).wait()

        def wait_write(b):
            pltpu.make_async_copy(bufs[b], out_hbm.at[pl.ds(0, CH)], sem_w[b]).wait()

        for b in range(nbuf):
            start_gather(b, b)

        @pl.loop(0, n_grp - 1)
        def _(grp):
            g0 = grp * nbuf
            for b in range(nbuf):
                wait_gather(b)
                start_write(b, g0 + b)
            for b in range(nbuf):
                wait_write(b)
                start_gather(b, g0 + nbuf + b)

        g0 = (n_grp - 1) * nbuf
        for b in range(nbuf):
            wait_gather(b)
            start_write(b, g0 + b)
        for b in range(nbuf):
            wait_write(b)

    return k(table, idx)


def _tc_add(a, b):
    n, dim = a.shape
    blk = 2048

    def body(a_ref, b_ref, o_ref):
        o_ref[...] = a_ref[...] + b_ref[...]

    return pl.pallas_call(
        body,
        grid=(n // blk,),
        in_specs=[
            pl.BlockSpec((blk, dim), lambda i: (i, 0)),
            pl.BlockSpec((blk, dim), lambda i: (i, 0)),
        ],
        out_specs=pl.BlockSpec((blk, dim), lambda i: (i, 0)),
        out_shape=jax.ShapeDtypeStruct((n, dim), jnp.float32),
    )(a, b)


def kernel(seqs, d, c, u, Wd, Wc, Wu):
    B, L, U = seqs.shape
    BL = B * L
    dv = Wd.shape[0]

    u2 = u.reshape(BL).astype(jnp.int32)
    dc_idx = jnp.stack(
        [d.astype(jnp.int32), c.astype(jnp.int32) + dv], axis=-1
    ).reshape(2 * BL)
    Wdc = jnp.concatenate([Wd, Wc], axis=0)

    ue = _gather_rows(Wu, u2, U)
    d_c = _gather_rows(Wdc, dc_idx, Wd.shape[1])
    seqs_u = _tc_add(seqs.reshape(BL, U), ue)

    return seqs_u.reshape(B, L, U), d_c.reshape(B, L, 2 * Wd.shape[1])


# speedup vs baseline: 1.3634x; 1.3634x over previous
import functools

import jax
import jax.numpy as jnp
from jax import lax
from jax.experimental import pallas as pl
from jax.experimental.pallas import tpu as pltpu
from jax.experimental.pallas import tpu_sc as plsc

NC, NS = 2, 16
NW = NC * NS


def _pad_idx(i2d, LP):
    B, L = i2d.shape
    return jnp.pad(i2d.astype(jnp.int32), ((0, 0), (0, LP - L))).reshape(B * LP)


def _sc_gather(tables, idxs, B, L, LP, dim, nbuf=4):
    nt = len(tables)
    rows_w = B // NW
    n_grp = rows_w // nbuf
    assert rows_w % nbuf == 0
    mesh = plsc.VectorSubcoreMesh(
        core_axis_name="c", subcore_axis_name="s", num_cores=NC, num_subcores=NS
    )
    cp = pltpu.CompilerParams(use_tc_tiling_on_sc=True)
    out_t = [jax.ShapeDtypeStruct((B, L, dim), jnp.float32)] * nt

    @functools.partial(
        pl.kernel,
        out_type=out_t,
        mesh=mesh,
        compiler_params=cp,
        scratch_types=[
            [pltpu.VMEM((rows_w * LP,), jnp.int32)] * nt,
            [[pltpu.VMEM((L, dim), jnp.float32)] * nbuf] * nt,
            [pltpu.SemaphoreType.DMA] * nbuf,
            [pltpu.SemaphoreType.DMA] * nbuf,
        ],
    )
    def k(*refs):
        t_hbm = refs[:nt]
        i_hbm = refs[nt : 2 * nt]
        o_hbm = refs[2 * nt : 3 * nt]
        idx_v = refs[3 * nt]
        bufs = refs[3 * nt + 1]
        sem_g = refs[3 * nt + 2]
        sem_w = refs[3 * nt + 3]

        wid = lax.axis_index("s") * NC + lax.axis_index("c")
        b0 = wid * rows_w

        for t in range(nt):
            pltpu.sync_copy(i_hbm[t].at[pl.ds(b0 * LP, rows_w * LP)], idx_v[t])

        def start_gather(s, j):
            for t in range(nt):
                pltpu.make_async_copy(
                    t_hbm[t].at[idx_v[t].at[pl.ds(j * LP, L)]],
                    bufs[t][s],
                    sem_g[s],
                ).start()

        def start_write(s, j):
            for t in range(nt):
                pltpu.make_async_copy(
                    bufs[t][s], o_hbm[t].at[b0 + j], sem_w[s]
                ).start()

        def wait_gather(s):
            for t in range(nt):
                pltpu.make_async_copy(
                    t_hbm[t].at[idx_v[t].at[pl.ds(0, L)]], bufs[t][s], sem_g[s]
                ).wait()

        def wait_write(s):
            for t in range(nt):
                pltpu.make_async_copy(bufs[t][s], o_hbm[t].at[0], sem_w[s]).wait()

        for s in range(nbuf):
            start_gather(s, s)

        @pl.loop(0, n_grp - 1)
        def _(grp):
            g0 = grp * nbuf
            for s in range(nbuf):
                wait_gather(s)
                start_write(s, g0 + s)
            for s in range(nbuf):
                wait_write(s)
                start_gather(s, g0 + nbuf + s)

        g0 = (n_grp - 1) * nbuf
        for s in range(nbuf):
            wait_gather(s)
            start_write(s, g0 + s)
        for s in range(nbuf):
            wait_write(s)

    return k(*tables, *idxs)


def _tc_add(a, b, blk=128):
    B, L, dim = a.shape

    def body(a_ref, b_ref, o_ref):
        o_ref[...] = a_ref[...] + b_ref[...]

    spec = pl.BlockSpec((blk, L, dim), lambda i: (i, 0, 0))
    return pl.pallas_call(
        body,
        grid=(B // blk,),
        in_specs=[spec, spec],
        out_specs=spec,
        out_shape=jax.ShapeDtypeStruct((B, L, dim), jnp.float32),
    )(a, b)


def kernel(seqs, d, c, u, Wd, Wc, Wu):
    B, L, U = seqs.shape
    LP = 56
    dd = Wd.shape[1]

    u_idx = _pad_idx(u.reshape(B, L), LP)
    d_idx = _pad_idx(d.reshape(B, L), LP)
    c_idx = _pad_idx(c.reshape(B, L), LP)
    WdP = jnp.pad(Wd, ((0, 0), (0, U - dd)))
    WcP = jnp.pad(Wc, ((0, 0), (U - dd, 0)))

    (ue,) = _sc_gather([Wu], [u_idx], B, L, LP, U)
    deP, ceP = _sc_gather([WdP, WcP], [d_idx, c_idx], B, L, LP, U)
    seqs_u = _tc_add(seqs, ue)
    d_c = _tc_add(deP, ceP)
    return seqs_u, d_c


# speedup vs baseline: 2.8006x; 2.0541x over previous
import functools

import jax
import jax.numpy as jnp
from jax import lax
from jax.experimental import pallas as pl
from jax.experimental.pallas import tpu as pltpu
from jax.experimental.pallas import tpu_sc as plsc

NC, NS = 2, 16
NW = NC * NS


def _sc_gather(table, idx, dim, ch=128, nbuf=5):
    n = idx.shape[0]
    per_w = n // NW
    n_ch = per_w // ch
    n_grp = n_ch // nbuf
    assert per_w % ch == 0 and n_ch % nbuf == 0
    mesh = plsc.VectorSubcoreMesh(
        core_axis_name="c", subcore_axis_name="s", num_cores=NC, num_subcores=NS
    )

    @functools.partial(
        pl.kernel,
        out_type=jax.ShapeDtypeStruct((n, dim), jnp.float32),
        mesh=mesh,
        scratch_types=[
            pltpu.VMEM((per_w,), jnp.int32),
            [pltpu.VMEM((ch, dim), jnp.float32)] * nbuf,
            [pltpu.SemaphoreType.DMA] * nbuf,
            [pltpu.SemaphoreType.DMA] * nbuf,
        ],
    )
    def k(table_hbm, idx_hbm, out_hbm, idx_v, bufs, sem_g, sem_w):
        wid = lax.axis_index("s") * NC + lax.axis_index("c")
        w_base = wid * per_w

        pltpu.sync_copy(idx_hbm.at[pl.ds(w_base, per_w)], idx_v)

        def start_gather(s, j):
            pltpu.make_async_copy(
                table_hbm.at[idx_v.at[pl.ds(j * ch, ch)]], bufs[s], sem_g[s]
            ).start()

        def start_write(s, j):
            pltpu.make_async_copy(
                bufs[s], out_hbm.at[pl.ds(w_base + j * ch, ch)], sem_w[s]
            ).start()

        def wait_gather(s):
            pltpu.make_async_copy(
                table_hbm.at[idx_v.at[pl.ds(0, ch)]], bufs[s], sem_g[s]
            ).wait()

        def wait_write(s):
            pltpu.make_async_copy(bufs[s], out_hbm.at[pl.ds(0, ch)], sem_w[s]).wait()

        for s in range(nbuf):
            start_gather(s, s)

        @pl.loop(0, n_grp - 1)
        def _(grp):
            g0 = grp * nbuf
            for s in range(nbuf):
                wait_gather(s)
                start_write(s, g0 + s)
            for s in range(nbuf):
                wait_write(s)
                start_gather(s, g0 + nbuf + s)

        g0 = (n_grp - 1) * nbuf
        for s in range(nbuf):
            wait_gather(s)
            start_write(s, g0 + s)
        for s in range(nbuf):
            wait_write(s)

    return k(table, idx)


def _sc_gather_add(ta, tb, ia, ib, dim, ch=64, nbuf=4):
    n = ia.shape[0]
    per_w = n // NW
    n_ch = per_w // ch
    n_grp = n_ch // nbuf
    assert per_w % ch == 0 and n_ch % nbuf == 0
    mesh = plsc.VectorSubcoreMesh(
        core_axis_name="c", subcore_axis_name="s", num_cores=NC, num_subcores=NS
    )

    @functools.partial(
        pl.kernel,
        out_type=jax.ShapeDtypeStruct((n, dim), jnp.float32),
        mesh=mesh,
        scratch_types=[
            [pltpu.VMEM((per_w,), jnp.int32)] * 2,
            [[pltpu.VMEM((ch, dim), jnp.float32)] * nbuf] * 3,
            [pltpu.SemaphoreType.DMA] * nbuf,
            [pltpu.SemaphoreType.DMA] * nbuf,
        ],
    )
    def k(ta_hbm, tb_hbm, ia_hbm, ib_hbm, o_hbm, idx_v, bufs, sem_g, sem_w):
        t_hbm = (ta_hbm, tb_hbm)
        i_hbm = (ia_hbm, ib_hbm)
        bufA, bufB, bufO = bufs
        wid = lax.axis_index("s") * NC + lax.axis_index("c")
        w_base = wid * per_w

        for t in range(2):
            pltpu.sync_copy(i_hbm[t].at[pl.ds(w_base, per_w)], idx_v[t])

        def start_gather(s, j):
            for t, buf in ((0, bufA), (1, bufB)):
                pltpu.make_async_copy(
                    t_hbm[t].at[idx_v[t].at[pl.ds(j * ch, ch)]], buf[s], sem_g[s]
                ).start()

        def wait_gather(s):
            for t, buf in ((0, bufA), (1, bufB)):
                pltpu.make_async_copy(
                    t_hbm[t].at[idx_v[t].at[pl.ds(0, ch)]], buf[s], sem_g[s]
                ).wait()

        def start_write(s, j):
            pltpu.make_async_copy(
                bufO[s], o_hbm.at[pl.ds(w_base + j * ch, ch)], sem_w[s]
            ).start()

        def wait_write(s):
            pltpu.make_async_copy(bufO[s], o_hbm.at[pl.ds(0, ch)], sem_w[s]).wait()

        def add(s):
            @pl.loop(0, ch)
            def _(r):
                for kk in range(dim // 16):
                    sl = pl.ds(kk * 16, 16)
                    bufO[s][r, sl] = bufA[s][r, sl] + bufB[s][r, sl]

        for s in range(nbuf):
            start_gather(s, s)

        @pl.loop(0, n_grp)
        def _(grp):
            g0 = grp * nbuf
            for s in range(nbuf):
                wait_gather(s)

                @pl.when(grp > 0)
                def _():
                    wait_write(s)

                add(s)
                start_write(s, g0 + s)
            for s in range(nbuf):

                @pl.when(grp < n_grp - 1)
                def _():
                    start_gather(s, g0 + nbuf + s)

        for s in range(nbuf):
            wait_write(s)

    return k(ta, tb, ia, ib)


def _tc_add(a, b, blk=2048):
    n, dim = a.shape

    def body(a_ref, b_ref, o_ref):
        o_ref[...] = a_ref[...] + b_ref[...]

    spec = pl.BlockSpec((blk, dim), lambda i: (i, 0))
    return pl.pallas_call(
        body,
        grid=(n // blk,),
        in_specs=[spec, spec],
        out_specs=spec,
        out_shape=jax.ShapeDtypeStruct((n, dim), jnp.float32),
    )(a, b)


def kernel(seqs, d, c, u, Wd, Wc, Wu):
    B, L, U = seqs.shape
    BL = B * L
    dd = Wd.shape[1]

    def t_flat(i2d):
        return i2d.reshape(B, L).astype(jnp.int32).T.reshape(BL)

    u_idx = t_flat(u)
    d_idx = t_flat(d)
    c_idx = t_flat(c)
    WdP = jnp.pad(Wd, ((0, 0), (0, U - dd)))
    WcP = jnp.pad(Wc, ((0, 0), (dd, 0)))

    ue = _sc_gather(Wu, u_idx, U)
    d_c = _sc_gather_add(WdP, WcP, d_idx, c_idx, U)
    seqs_t = seqs.transpose(1, 0, 2).reshape(BL, U)
    seqs_u = _tc_add(seqs_t, ue)

    def untranspose(flat):
        return flat.reshape(L, B, U).transpose(1, 0, 2)

    return untranspose(seqs_u), untranspose(d_c)


# speedup vs baseline: 2.8181x; 1.0062x over previous
import functools

import jax
import jax.numpy as jnp
from jax import lax
from jax.experimental import pallas as pl
from jax.experimental.pallas import tpu as pltpu
from jax.experimental.pallas import tpu_sc as plsc

NC, NS = 2, 16
NW = NC * NS


def _sc_gather(table, idx, dim, ch=128, nbuf=5):
    n = idx.shape[0]
    per_w = n // NW
    n_ch = per_w // ch
    n_grp = n_ch // nbuf
    assert per_w % ch == 0 and n_ch % nbuf == 0
    mesh = plsc.VectorSubcoreMesh(
        core_axis_name="c", subcore_axis_name="s", num_cores=NC, num_subcores=NS
    )

    @functools.partial(
        pl.kernel,
        out_type=jax.ShapeDtypeStruct((n, dim), jnp.float32),
        mesh=mesh,
        scratch_types=[
            pltpu.VMEM((per_w,), jnp.int32),
            [pltpu.VMEM((ch, dim), jnp.float32)] * nbuf,
            [pltpu.SemaphoreType.DMA] * nbuf,
            [pltpu.SemaphoreType.DMA] * nbuf,
        ],
    )
    def k(table_hbm, idx_hbm, out_hbm, idx_v, bufs, sem_g, sem_w):
        wid = lax.axis_index("s") * NC + lax.axis_index("c")
        w_base = wid * per_w

        pltpu.sync_copy(idx_hbm.at[pl.ds(w_base, per_w)], idx_v)

        def start_gather(s, j):
            pltpu.make_async_copy(
                table_hbm.at[idx_v.at[pl.ds(j * ch, ch)]], bufs[s], sem_g[s]
            ).start()

        def start_write(s, j):
            pltpu.make_async_copy(
                bufs[s], out_hbm.at[pl.ds(w_base + j * ch, ch)], sem_w[s]
            ).start()

        def wait_gather(s):
            pltpu.make_async_copy(
                table_hbm.at[idx_v.at[pl.ds(0, ch)]], bufs[s], sem_g[s]
            ).wait()

        def wait_write(s):
            pltpu.make_async_copy(bufs[s], out_hbm.at[pl.ds(0, ch)], sem_w[s]).wait()

        for s in range(nbuf):
            start_gather(s, s)

        @pl.loop(0, n_grp - 1)
        def _(grp):
            g0 = grp * nbuf
            for s in range(nbuf):
                wait_gather(s)
                start_write(s, g0 + s)
            for s in range(nbuf):
                wait_write(s)
                start_gather(s, g0 + nbuf + s)

        g0 = (n_grp - 1) * nbuf
        for s in range(nbuf):
            wait_gather(s)
            start_write(s, g0 + s)
        for s in range(nbuf):
            wait_write(s)

    return k(table, idx)


def _sc_gather_add(ta, tb, ia, ib, dim, ch=64, nbuf=4):
    n = ia.shape[0]
    per_w = n // NW
    n_ch = per_w // ch
    n_grp = n_ch // nbuf
    assert per_w % ch == 0 and n_ch % nbuf == 0
    mesh = plsc.VectorSubcoreMesh(
        core_axis_name="c", subcore_axis_name="s", num_cores=NC, num_subcores=NS
    )

    @functools.partial(
        pl.kernel,
        out_type=jax.ShapeDtypeStruct((n, dim), jnp.float32),
        mesh=mesh,
        scratch_types=[
            [pltpu.VMEM((per_w,), jnp.int32)] * 2,
            [[pltpu.VMEM((ch, dim), jnp.float32)] * nbuf] * 3,
            [pltpu.SemaphoreType.DMA] * nbuf,
            [pltpu.SemaphoreType.DMA] * nbuf,
        ],
    )
    def k(ta_hbm, tb_hbm, ia_hbm, ib_hbm, o_hbm, idx_v, bufs, sem_g, sem_w):
        t_hbm = (ta_hbm, tb_hbm)
        i_hbm = (ia_hbm, ib_hbm)
        bufA, bufB, bufO = bufs
        wid = lax.axis_index("s") * NC + lax.axis_index("c")
        w_base = wid * per_w

        for t in range(2):
            pltpu.sync_copy(i_hbm[t].at[pl.ds(w_base, per_w)], idx_v[t])

        def start_gather(s, j):
            for t, buf in ((0, bufA), (1, bufB)):
                pltpu.make_async_copy(
                    t_hbm[t].at[idx_v[t].at[pl.ds(j * ch, ch)]], buf[s], sem_g[s]
                ).start()

        def wait_gather(s):
            for t, buf in ((0, bufA), (1, bufB)):
                pltpu.make_async_copy(
                    t_hbm[t].at[idx_v[t].at[pl.ds(0, ch)]], buf[s], sem_g[s]
                ).wait()

        def start_write(s, j):
            pltpu.make_async_copy(
                bufO[s], o_hbm.at[pl.ds(w_base + j * ch, ch)], sem_w[s]
            ).start()

        def wait_write(s):
            pltpu.make_async_copy(bufO[s], o_hbm.at[pl.ds(0, ch)], sem_w[s]).wait()

        def add(s):
            @pl.loop(0, ch)
            def _(r):
                for kk in range(dim // 16):
                    sl = pl.ds(kk * 16, 16)
                    bufO[s][r, sl] = bufA[s][r, sl] + bufB[s][r, sl]

        for s in range(nbuf):
            start_gather(s, s)

        @pl.loop(0, n_grp)
        def _(grp):
            g0 = grp * nbuf
            for s in range(nbuf):
                wait_gather(s)

                @pl.when(grp > 0)
                def _():
                    wait_write(s)

                add(s)
                start_write(s, g0 + s)

                @pl.when(grp < n_grp - 1)
                def _():
                    start_gather(s, g0 + nbuf + s)

        for s in range(nbuf):
            wait_write(s)

    return k(ta, tb, ia, ib)


def _tc_add(a, b, blk=2048):
    n, dim = a.shape

    def body(a_ref, b_ref, o_ref):
        o_ref[...] = a_ref[...] + b_ref[...]

    spec = pl.BlockSpec((blk, dim), lambda i: (i, 0))
    return pl.pallas_call(
        body,
        grid=(n // blk,),
        in_specs=[spec, spec],
        out_specs=spec,
        out_shape=jax.ShapeDtypeStruct((n, dim), jnp.float32),
    )(a, b)


def kernel(seqs, d, c, u, Wd, Wc, Wu):
    B, L, U = seqs.shape
    BL = B * L
    dd = Wd.shape[1]

    def t_flat(i2d):
        return i2d.reshape(B, L).astype(jnp.int32).T.reshape(BL)

    u_idx = t_flat(u)
    d_idx = t_flat(d)
    c_idx = t_flat(c)
    WdP = jnp.pad(Wd, ((0, 0), (0, U - dd)))
    WcP = jnp.pad(Wc, ((0, 0), (dd, 0)))

    ue = _sc_gather(Wu, u_idx, U)
    d_c = _sc_gather_add(WdP, WcP, d_idx, c_idx, U)
    seqs_t = seqs.transpose(1, 0, 2).reshape(BL, U)
    seqs_u = _tc_add(seqs_t, ue)

    def untranspose(flat):
        return flat.reshape(L, B, U).transpose(1, 0, 2)

    return untranspose(seqs_u), untranspose(d_c)
